# baseline (speedup 1.0000x reference)
import functools

import jax
import jax.numpy as jnp
from jax import lax
from jax.experimental import pallas as pl
from jax.experimental.pallas import tpu as pltpu
from jax.experimental.pallas import tpu_sc as plsc

N = 10000
E = 320000
F = 128
C = 16

NC = 2
NS = 16
L = 16
NW = NC * NS

K = 80
CHUNKS_PER_TILE = E // (NW * K)
NP = 10240
NPT = NP // NS

BN = 512
GRID = -(-N // BN)

_mesh = plsc.VectorSubcoreMesh(core_axis_name="c", subcore_axis_name="s")

_GDN = lax.GatherDimensionNumbers(
    offset_dims=(), collapsed_slice_dims=(0,), start_index_map=(0,))


def _take16(vec, k):
    idx = jnp.zeros((L, 1), jnp.int32) + k
    return lax.gather(vec, idx, dimension_numbers=_GDN,
                      slice_sizes=(1,),
                      mode=lax.GatherScatterMode.PROMISE_IN_BOUNDS)


def _rsqrt16(d):
    i = lax.bitcast_convert_type(d, jnp.int32)
    i = 0x5F3759DF - lax.shift_right_logical(i, 1)
    y = lax.bitcast_convert_type(i, jnp.float32)
    for _ in range(3):
        y = y * (1.5 - 0.5 * d * y * y)
    return y


@functools.partial(
    pl.kernel,
    out_type=[
        jax.ShapeDtypeStruct((NC, NP, C), jnp.float32),
        jax.ShapeDtypeStruct((NP,), jnp.float32),
    ],
    mesh=_mesh,
    compiler_params=pltpu.CompilerParams(use_tc_tiling_on_sc=False),
    scratch_types=[
        pltpu.VMEM((CHUNKS_PER_TILE, K), jnp.int32),
        pltpu.VMEM((CHUNKS_PER_TILE, K), jnp.int32),
        pltpu.VMEM((CHUNKS_PER_TILE, K), jnp.float32),
        pltpu.VMEM((NPT, C), jnp.float32),
        pltpu.VMEM((NPT,), jnp.float32),
        pltpu.VMEM((K, C), jnp.float32),
        pltpu.VMEM((K, C), jnp.float32),
        pltpu.VMEM_SHARED((NP, C), jnp.float32),
        pltpu.VMEM_SHARED((NP, C), jnp.float32),
        pltpu.VMEM_SHARED((NP,), jnp.float32),
        pltpu.SemaphoreType.DMA,
        pltpu.SemaphoreType.DMA,
        pltpu.SemaphoreType.DMA,
        pltpu.SemaphoreType.DMA,
        pltpu.SemaphoreType.DMA,
    ],
)
def _edge_kernel(row_hbm, col_hbm, ew_hbm, h_hbm, sp_hbm, deg_hbm,
                 row_v, col_v, ew_v, gbuf, d0_v, rows0, rows1,
                 g_sh, s_sh, deg_sh, sem0, sem1, ssem0, ssem1, dsem):
    c = lax.axis_index("c")
    s = lax.axis_index("s")
    wid = c * NS + s

    def zero_body(i, _):
        d0_v[pl.ds(i * L, L)] = jnp.zeros((L,), jnp.float32)
        return 0
    lax.fori_loop(0, NPT // L, zero_body, 0)
    pltpu.sync_copy(d0_v, deg_sh.at[pl.ds(s * NPT, NPT)])
    pltpu.sync_copy(row_hbm.at[wid], row_v)
    pltpu.sync_copy(h_hbm.at[pl.ds(s * NPT, NPT)], gbuf)
    plsc.subcore_barrier()

    for half in range(2):
        w = s * 2 + half
        pltpu.sync_copy(col_hbm.at[w], col_v)
        pltpu.sync_copy(ew_hbm.at[w], ew_v)

        def dscat_body(i, _):
            pltpu.async_copy(ew_v.at[i], deg_sh.at[col_v.at[i]], dsem,
                             add=True)
            return 0
        lax.fori_loop(0, CHUNKS_PER_TILE, dscat_body, 0)

        def ddrain_body(i, _):
            pltpu.make_async_copy(ew_v.at[0], deg_sh.at[col_v.at[0]],
                                  dsem).wait()
            return 0
        lax.fori_loop(0, CHUNKS_PER_TILE, ddrain_body, 0)
    plsc.subcore_barrier()

    pltpu.sync_copy(deg_sh.at[pl.ds(s * NPT, NPT)], d0_v)

    def g_body(v, _):
        d = 1.0 + d0_v[pl.ds(v * L, L)]
        dv = _rsqrt16(d)
        for k2 in range(L):
            n = v * L + k2
            gbuf[n, :] = gbuf[n, :] * _take16(dv, k2)
        return 0
    lax.fori_loop(0, NPT // L, g_body, 0)

    pltpu.sync_copy(gbuf, g_sh.at[pl.ds(s * NPT, NPT)])
    pltpu.sync_copy(gbuf, s_sh.at[pl.ds(s * NPT, NPT)])

    @pl.when(c == 0)
    def _deg_out():
        pltpu.sync_copy(d0_v, deg_hbm.at[pl.ds(s * NPT, NPT)])

    pltpu.sync_copy(col_hbm.at[wid], col_v)
    pltpu.sync_copy(ew_hbm.at[wid], ew_v)
    plsc.subcore_barrier()

    rows = (rows0, rows1)
    sems = (sem0, sem1)
    ssems = (ssem0, ssem1)

    def _scale(i, buf):
        for j in range(K // L):
            ewv = ew_v[i, pl.ds(j * L, L)]
            for k2 in range(L):
                e = j * L + k2
                buf[e, :] = buf[e, :] * _take16(ewv, k2)

    pltpu.async_copy(g_sh.at[row_v.at[0]], rows0, sem0)
    pltpu.async_copy(g_sh.at[row_v.at[1]], rows1, sem1)
    pltpu.make_async_copy(g_sh.at[row_v.at[0]], rows0, sem0).wait()
    _scale(0, rows0)
    pltpu.async_copy(rows0, s_sh.at[col_v.at[0]], ssem0, add=True)

    def pair_body(i2, _):
        for b in (1, 0):
            i = i2 * 2 + (2 - b)
            ob = 1 - b
            pltpu.make_async_copy(rows[ob], s_sh.at[col_v.at[0]], ssems[ob]).wait()
            pltpu.async_copy(g_sh.at[row_v.at[i + 1]], rows[ob], sems[ob])
            pltpu.make_async_copy(g_sh.at[row_v.at[i]], rows[b], sems[b]).wait()
            _scale(i, rows[b])
            pltpu.async_copy(rows[b], s_sh.at[col_v.at[i]], ssems[b], add=True)
        return 0
    lax.fori_loop(0, (CHUNKS_PER_TILE - 3) // 2, pair_body, 0)

    i = CHUNKS_PER_TILE - 2
    pltpu.make_async_copy(rows0, s_sh.at[col_v.at[0]], ssem0).wait()
    pltpu.async_copy(g_sh.at[row_v.at[i + 1]], rows0, sem0)
    pltpu.make_async_copy(g_sh.at[row_v.at[i]], rows1, sem1).wait()
    _scale(i, rows1)
    pltpu.async_copy(rows1, s_sh.at[col_v.at[i]], ssem1, add=True)
    i = CHUNKS_PER_TILE - 1
    pltpu.make_async_copy(g_sh.at[row_v.at[i]], rows0, sem0).wait()
    _scale(i, rows0)
    pltpu.async_copy(rows0, s_sh.at[col_v.at[i]], ssem0, add=True)
    pltpu.make_async_copy(rows1, s_sh.at[col_v.at[0]], ssem1).wait()
    pltpu.make_async_copy(rows0, s_sh.at[col_v.at[0]], ssem0).wait()

    plsc.subcore_barrier()
    pltpu.sync_copy(s_sh.at[pl.ds(s * NPT, NPT)],
                    sp_hbm.at[c, pl.ds(s * NPT, NPT)])


def _tc1_body(x_ref, w_ref, h_ref):
    h_ref[...] = jnp.dot(x_ref[...], w_ref[...],
                         preferred_element_type=jnp.float32)


def _tc1(x, W):
    return pl.pallas_call(
        _tc1_body,
        grid=(GRID,),
        in_specs=[
            pl.BlockSpec((BN, F), lambda i: (i, 0)),
            pl.BlockSpec((F, C), lambda i: (0, 0)),
        ],
        out_specs=pl.BlockSpec((BN, C), lambda i: (i, 0)),
        out_shape=jax.ShapeDtypeStruct((NP, C), jnp.float32),
    )(x, W)


def _tc2_body(sp_ref, h_ref, deg_ref, b_ref, out_ref):
    dinv = lax.rsqrt(1.0 + deg_ref[...])
    t = sp_ref[0] + sp_ref[1] - h_ref[...] * dinv[:, None]
    z = t * dinv[:, None] + b_ref[...]
    out_ref[...] = jax.nn.sigmoid(z)


def _tc2(sp, h, deg, b):
    return pl.pallas_call(
        _tc2_body,
        grid=(GRID,),
        in_specs=[
            pl.BlockSpec((NC, BN, C), lambda i: (0, i, 0)),
            pl.BlockSpec((BN, C), lambda i: (i, 0)),
            pl.BlockSpec((BN,), lambda i: (i,)),
            pl.BlockSpec((1, C), lambda i: (0, 0)),
        ],
        out_specs=pl.BlockSpec((BN, C), lambda i: (i, 0)),
        out_shape=jax.ShapeDtypeStruct((N, C), jnp.float32),
    )(sp, h, deg, b)


@jax.jit
def kernel(x, edge_index, edge_weight, W, b):
    row = edge_index[0].reshape(NW, CHUNKS_PER_TILE, K)
    col = edge_index[1].reshape(NW, CHUNKS_PER_TILE, K)
    ew = edge_weight.reshape(NW, CHUNKS_PER_TILE, K)

    h = _tc1(x, W)
    sp, deg = _edge_kernel(row, col, ew, h)
    out = _tc2(sp, h, deg, b.reshape(1, C))
    return out


# speedup vs baseline: 1.1683x; 1.1683x over previous
import functools

import jax
import jax.numpy as jnp
from jax import lax
from jax.experimental import pallas as pl
from jax.experimental.pallas import tpu as pltpu
from jax.experimental.pallas import tpu_sc as plsc

N = 10000
E = 320000
F = 128
C = 16

NC = 2
NS = 16
L = 16
NW = NC * NS

K = 80
CHUNKS_PER_TILE = E // (NW * K)
NP = 10240
NPT = NP // NS

BN = 512
GRID = -(-N // BN)
BM = 2048

_mesh = plsc.VectorSubcoreMesh(core_axis_name="c", subcore_axis_name="s")

_GDN = lax.GatherDimensionNumbers(
    offset_dims=(), collapsed_slice_dims=(0,), start_index_map=(0,))


def _take16(vec, k):
    idx = jnp.zeros((L, 1), jnp.int32) + k
    return lax.gather(vec, idx, dimension_numbers=_GDN,
                      slice_sizes=(1,),
                      mode=lax.GatherScatterMode.PROMISE_IN_BOUNDS)


@functools.partial(
    pl.kernel,
    out_type=jax.ShapeDtypeStruct((NC, NP), jnp.float32),
    mesh=_mesh,
    scratch_types=[
        pltpu.VMEM((CHUNKS_PER_TILE, K), jnp.int32),
        pltpu.VMEM((CHUNKS_PER_TILE, K), jnp.float32),
        pltpu.VMEM((NPT,), jnp.float32),
        pltpu.VMEM_SHARED((NP,), jnp.float32),
        pltpu.SemaphoreType.DMA,
    ],
)
def _deg_kernel(col_hbm, ew_hbm, degp_hbm, col_v, ew_v, zbuf, deg_sh, dsem):
    c = lax.axis_index("c")
    s = lax.axis_index("s")
    wid = c * NS + s

    pltpu.sync_copy(col_hbm.at[wid], col_v)
    pltpu.sync_copy(ew_hbm.at[wid], ew_v)

    def zero_body(i, _):
        zbuf[pl.ds(i * L, L)] = jnp.zeros((L,), jnp.float32)
        return 0
    lax.fori_loop(0, NPT // L, zero_body, 0)
    pltpu.sync_copy(zbuf, deg_sh.at[pl.ds(s * NPT, NPT)])
    plsc.subcore_barrier()

    def chunk_body(i, _):
        pltpu.async_copy(ew_v.at[i], deg_sh.at[col_v.at[i]], dsem, add=True)
        return 0
    lax.fori_loop(0, CHUNKS_PER_TILE, chunk_body, 0)

    def drain_body(i, _):
        pltpu.make_async_copy(ew_v.at[0], deg_sh.at[col_v.at[0]], dsem).wait()
        return 0
    lax.fori_loop(0, CHUNKS_PER_TILE, drain_body, 0)

    plsc.subcore_barrier()
    pltpu.sync_copy(deg_sh.at[pl.ds(s * NPT, NPT)],
                    degp_hbm.at[c, pl.ds(s * NPT, NPT)])


def _rsqrt16(d):
    i = lax.bitcast_convert_type(d, jnp.int32)
    i = 0x5F3759DF - lax.shift_right_logical(i, 1)
    y = lax.bitcast_convert_type(i, jnp.float32)
    for _ in range(3):
        y = y * (1.5 - 0.5 * d * y * y)
    return y


@functools.partial(
    pl.kernel,
    out_type=jax.ShapeDtypeStruct((NC, NP, C), jnp.float32),
    mesh=_mesh,
    compiler_params=pltpu.CompilerParams(use_tc_tiling_on_sc=False),
    scratch_types=[
        pltpu.VMEM((CHUNKS_PER_TILE, K), jnp.int32),
        pltpu.VMEM((CHUNKS_PER_TILE, K), jnp.int32),
        pltpu.VMEM((CHUNKS_PER_TILE, K), jnp.float32),
        pltpu.VMEM((NPT, C), jnp.float32),
        pltpu.VMEM((NPT,), jnp.float32),
        pltpu.VMEM((NPT,), jnp.float32),
        pltpu.VMEM((K, C), jnp.float32),
        pltpu.VMEM((K, C), jnp.float32),
        pltpu.VMEM_SHARED((NP, C), jnp.float32),
        pltpu.VMEM_SHARED((NP, C), jnp.float32),
        pltpu.SemaphoreType.DMA,
        pltpu.SemaphoreType.DMA,
        pltpu.SemaphoreType.DMA,
        pltpu.SemaphoreType.DMA,
    ],
)
def _edge_kernel(row_hbm, col_hbm, ew_hbm, h_hbm, degp_hbm, sp_hbm,
                 row_v, col_v, ew_v, gbuf, d0_v, d1_v, rows0, rows1,
                 g_sh, s_sh, sem0, sem1, ssem0, ssem1):
    c = lax.axis_index("c")
    s = lax.axis_index("s")
    wid = c * NS + s

    pltpu.sync_copy(row_hbm.at[wid], row_v)
    pltpu.sync_copy(col_hbm.at[wid], col_v)
    pltpu.sync_copy(ew_hbm.at[wid], ew_v)

    pltpu.sync_copy(h_hbm.at[pl.ds(s * NPT, NPT)], gbuf)
    pltpu.sync_copy(degp_hbm.at[0, pl.ds(s * NPT, NPT)], d0_v)
    pltpu.sync_copy(degp_hbm.at[1, pl.ds(s * NPT, NPT)], d1_v)

    def g_body(v, _):
        d = 1.0 + d0_v[pl.ds(v * L, L)] + d1_v[pl.ds(v * L, L)]
        dv = _rsqrt16(d)
        for k2 in range(L):
            n = v * L + k2
            gbuf[n, :] = gbuf[n, :] * _take16(dv, k2)
        return 0
    lax.fori_loop(0, NPT // L, g_body, 0)

    pltpu.sync_copy(gbuf, g_sh.at[pl.ds(s * NPT, NPT)])
    pltpu.sync_copy(gbuf, s_sh.at[pl.ds(s * NPT, NPT)])
    plsc.subcore_barrier()

    rows = (rows0, rows1)
    sems = (sem0, sem1)
    ssems = (ssem0, ssem1)

    def _scale(i, buf):
        for j in range(K // L):
            ewv = ew_v[i, pl.ds(j * L, L)]
            for k2 in range(L):
                e = j * L + k2
                buf[e, :] = buf[e, :] * _take16(ewv, k2)

    pltpu.async_copy(g_sh.at[row_v.at[0]], rows0, sem0)
    pltpu.async_copy(g_sh.at[row_v.at[1]], rows1, sem1)
    pltpu.make_async_copy(g_sh.at[row_v.at[0]], rows0, sem0).wait()
    _scale(0, rows0)
    pltpu.async_copy(rows0, s_sh.at[col_v.at[0]], ssem0, add=True)

    def pair_body(i2, _):
        for b in (1, 0):
            i = i2 * 2 + (2 - b)
            ob = 1 - b
            pltpu.make_async_copy(rows[ob], s_sh.at[col_v.at[0]], ssems[ob]).wait()
            pltpu.async_copy(g_sh.at[row_v.at[i + 1]], rows[ob], sems[ob])
            pltpu.make_async_copy(g_sh.at[row_v.at[i]], rows[b], sems[b]).wait()
            _scale(i, rows[b])
            pltpu.async_copy(rows[b], s_sh.at[col_v.at[i]], ssems[b], add=True)
        return 0
    lax.fori_loop(0, (CHUNKS_PER_TILE - 3) // 2, pair_body, 0)

    i = CHUNKS_PER_TILE - 2
    pltpu.make_async_copy(rows0, s_sh.at[col_v.at[0]], ssem0).wait()
    pltpu.async_copy(g_sh.at[row_v.at[i + 1]], rows0, sem0)
    pltpu.make_async_copy(g_sh.at[row_v.at[i]], rows1, sem1).wait()
    _scale(i, rows1)
    pltpu.async_copy(rows1, s_sh.at[col_v.at[i]], ssem1, add=True)
    i = CHUNKS_PER_TILE - 1
    pltpu.make_async_copy(g_sh.at[row_v.at[i]], rows0, sem0).wait()
    _scale(i, rows0)
    pltpu.async_copy(rows0, s_sh.at[col_v.at[i]], ssem0, add=True)
    pltpu.make_async_copy(rows1, s_sh.at[col_v.at[0]], ssem1).wait()
    pltpu.make_async_copy(rows0, s_sh.at[col_v.at[0]], ssem0).wait()

    plsc.subcore_barrier()
    pltpu.sync_copy(s_sh.at[pl.ds(s * NPT, NPT)],
                    sp_hbm.at[c, pl.ds(s * NPT, NPT)])


def _tc1_body(x_ref, w_ref, h_ref):
    h_ref[...] = jnp.dot(x_ref[...], w_ref[...],
                         preferred_element_type=jnp.float32)


def _tc1(x, W):
    return pl.pallas_call(
        _tc1_body,
        grid=(NP // BM,),
        in_specs=[
            pl.BlockSpec((BM, F), lambda i: (i, 0)),
            pl.BlockSpec((F, C), lambda i: (0, 0)),
        ],
        out_specs=pl.BlockSpec((BM, C), lambda i: (i, 0)),
        out_shape=jax.ShapeDtypeStruct((NP, C), jnp.float32),
    )(x, W)


def _tc2_body(sp_ref, h_ref, degp_ref, b_ref, out_ref):
    dinv = lax.rsqrt(1.0 + degp_ref[0] + degp_ref[1])
    t = sp_ref[0] + sp_ref[1] - h_ref[...] * dinv[:, None]
    z = t * dinv[:, None] + b_ref[...]
    out_ref[...] = jax.nn.sigmoid(z)


def _tc2(sp, h, degp, b):
    return pl.pallas_call(
        _tc2_body,
        grid=(NP // BM,),
        in_specs=[
            pl.BlockSpec((NC, BM, C), lambda i: (0, i, 0)),
            pl.BlockSpec((BM, C), lambda i: (i, 0)),
            pl.BlockSpec((NC, BM), lambda i: (0, i)),
            pl.BlockSpec((1, C), lambda i: (0, 0)),
        ],
        out_specs=pl.BlockSpec((BM, C), lambda i: (i, 0)),
        out_shape=jax.ShapeDtypeStruct((N, C), jnp.float32),
    )(sp, h, degp, b)


@jax.jit
def kernel(x, edge_index, edge_weight, W, b):
    row = edge_index[0].reshape(NW, CHUNKS_PER_TILE, K)
    col = edge_index[1].reshape(NW, CHUNKS_PER_TILE, K)
    ew = edge_weight.reshape(NW, CHUNKS_PER_TILE, K)

    degp = _deg_kernel(col, ew)
    h = _tc1(x, W)
    sp = _edge_kernel(row, col, ew, h, degp)
    out = _tc2(sp, h, degp, b.reshape(1, C))
    return out


# speedup vs baseline: 1.2592x; 1.0778x over previous
import functools

import jax
import jax.numpy as jnp
from jax import lax
from jax.experimental import pallas as pl
from jax.experimental.pallas import tpu as pltpu
from jax.experimental.pallas import tpu_sc as plsc

N = 10000
E = 320000
F = 128
C = 16

NC = 2
NS = 16
L = 16
NW = NC * NS

K = 80
CHUNKS_PER_TILE = E // (NW * K)
NP = 10240
NPT = NP // NS

BN = 512
GRID = -(-N // BN)
BM = 2048

_mesh = plsc.VectorSubcoreMesh(core_axis_name="c", subcore_axis_name="s")

_GDN = lax.GatherDimensionNumbers(
    offset_dims=(), collapsed_slice_dims=(0,), start_index_map=(0,))


def _take16(vec, k):
    idx = jnp.zeros((L, 1), jnp.int32) + k
    return lax.gather(vec, idx, dimension_numbers=_GDN,
                      slice_sizes=(1,),
                      mode=lax.GatherScatterMode.PROMISE_IN_BOUNDS)


@functools.partial(
    pl.kernel,
    out_type=jax.ShapeDtypeStruct((NC, NP), jnp.float32),
    mesh=_mesh,
    compiler_params=pltpu.CompilerParams(use_tc_tiling_on_sc=False),
    scratch_types=[
        pltpu.VMEM((CHUNKS_PER_TILE, K), jnp.int32),
        pltpu.VMEM((CHUNKS_PER_TILE, K), jnp.float32),
        pltpu.VMEM((NPT,), jnp.float32),
        pltpu.VMEM_SHARED((NP,), jnp.float32),
        pltpu.SemaphoreType.DMA,
    ],
)
def _deg_kernel(col_hbm, ew_hbm, degp_hbm, col_v, ew_v, zbuf, deg_sh, dsem):
    c = lax.axis_index("c")
    s = lax.axis_index("s")
    wid = c * NS + s

    pltpu.sync_copy(col_hbm.at[wid], col_v)
    pltpu.sync_copy(ew_hbm.at[wid], ew_v)

    def zero_body(i, _):
        zbuf[pl.ds(i * L, L)] = jnp.zeros((L,), jnp.float32)
        return 0
    lax.fori_loop(0, NPT // L, zero_body, 0)
    pltpu.sync_copy(zbuf, deg_sh.at[pl.ds(s * NPT, NPT)])
    plsc.subcore_barrier()

    def chunk_body(i, _):
        pltpu.async_copy(ew_v.at[i], deg_sh.at[col_v.at[i]], dsem, add=True)
        return 0
    lax.fori_loop(0, CHUNKS_PER_TILE, chunk_body, 0)

    def drain_body(i, _):
        pltpu.make_async_copy(ew_v.at[0], deg_sh.at[col_v.at[0]], dsem).wait()
        return 0
    lax.fori_loop(0, CHUNKS_PER_TILE, drain_body, 0)

    plsc.subcore_barrier()
    pltpu.sync_copy(deg_sh.at[pl.ds(s * NPT, NPT)],
                    degp_hbm.at[c, pl.ds(s * NPT, NPT)])


def _rsqrt16(d):
    i = lax.bitcast_convert_type(d, jnp.int32)
    i = 0x5F3759DF - lax.shift_right_logical(i, 1)
    y = lax.bitcast_convert_type(i, jnp.float32)
    for _ in range(3):
        y = y * (1.5 - 0.5 * d * y * y)
    return y


@functools.partial(
    pl.kernel,
    out_type=jax.ShapeDtypeStruct((NC, NP, C), jnp.float32),
    mesh=_mesh,
    compiler_params=pltpu.CompilerParams(use_tc_tiling_on_sc=False),
    scratch_types=[
        pltpu.VMEM((CHUNKS_PER_TILE, K), jnp.int32),
        pltpu.VMEM((CHUNKS_PER_TILE, K), jnp.int32),
        pltpu.VMEM((CHUNKS_PER_TILE, K), jnp.float32),
        pltpu.VMEM((NPT, C), jnp.float32),
        pltpu.VMEM((NPT,), jnp.float32),
        pltpu.VMEM((NPT,), jnp.float32),
        pltpu.VMEM((K, C), jnp.float32),
        pltpu.VMEM((K, C), jnp.float32),
        pltpu.VMEM_SHARED((NP, C), jnp.float32),
        pltpu.VMEM_SHARED((NP, C), jnp.float32),
        pltpu.SemaphoreType.DMA,
        pltpu.SemaphoreType.DMA,
        pltpu.SemaphoreType.DMA,
        pltpu.SemaphoreType.DMA,
    ],
)
def _edge_kernel(row_hbm, col_hbm, ew_hbm, h_hbm, degp_hbm, sp_hbm,
                 row_v, col_v, ew_v, gbuf, d0_v, d1_v, rows0, rows1,
                 g_sh, s_sh, sem0, sem1, ssem0, ssem1):
    c = lax.axis_index("c")
    s = lax.axis_index("s")
    wid = c * NS + s

    pltpu.sync_copy(row_hbm.at[wid], row_v)
    pltpu.sync_copy(col_hbm.at[wid], col_v)
    pltpu.sync_copy(ew_hbm.at[wid], ew_v)

    pltpu.sync_copy(h_hbm.at[pl.ds(s * NPT, NPT)], gbuf)
    pltpu.sync_copy(degp_hbm.at[0, pl.ds(s * NPT, NPT)], d0_v)
    pltpu.sync_copy(degp_hbm.at[1, pl.ds(s * NPT, NPT)], d1_v)

    def g_body(v, _):
        d = 1.0 + d0_v[pl.ds(v * L, L)] + d1_v[pl.ds(v * L, L)]
        dv = _rsqrt16(d)
        for k2 in range(L):
            n = v * L + k2
            gbuf[n, :] = gbuf[n, :] * _take16(dv, k2)
        return 0
    lax.fori_loop(0, NPT // L, g_body, 0)

    pltpu.sync_copy(gbuf, g_sh.at[pl.ds(s * NPT, NPT)])
    pltpu.sync_copy(gbuf, s_sh.at[pl.ds(s * NPT, NPT)])
    plsc.subcore_barrier()

    rows = (rows0, rows1)
    sems = (sem0, sem1)
    ssems = (ssem0, ssem1)

    def _scale(i, buf):
        for j in range(K // L):
            ewv = ew_v[i, pl.ds(j * L, L)]
            for k2 in range(L):
                e = j * L + k2
                buf[e, :] = buf[e, :] * _take16(ewv, k2)

    pltpu.async_copy(g_sh.at[row_v.at[0]], rows0, sem0)
    pltpu.async_copy(g_sh.at[row_v.at[1]], rows1, sem1)
    pltpu.make_async_copy(g_sh.at[row_v.at[0]], rows0, sem0).wait()
    _scale(0, rows0)
    pltpu.async_copy(rows0, s_sh.at[col_v.at[0]], ssem0, add=True)

    def pair_body(i2, _):
        for b in (1, 0):
            i = i2 * 2 + (2 - b)
            ob = 1 - b
            pltpu.make_async_copy(rows[ob], s_sh.at[col_v.at[0]], ssems[ob]).wait()
            pltpu.async_copy(g_sh.at[row_v.at[i + 1]], rows[ob], sems[ob])
            pltpu.make_async_copy(g_sh.at[row_v.at[i]], rows[b], sems[b]).wait()
            _scale(i, rows[b])
            pltpu.async_copy(rows[b], s_sh.at[col_v.at[i]], ssems[b], add=True)
        return 0
    lax.fori_loop(0, (CHUNKS_PER_TILE - 3) // 2, pair_body, 0)

    i = CHUNKS_PER_TILE - 2
    pltpu.make_async_copy(rows0, s_sh.at[col_v.at[0]], ssem0).wait()
    pltpu.async_copy(g_sh.at[row_v.at[i + 1]], rows0, sem0)
    pltpu.make_async_copy(g_sh.at[row_v.at[i]], rows1, sem1).wait()
    _scale(i, rows1)
    pltpu.async_copy(rows1, s_sh.at[col_v.at[i]], ssem1, add=True)
    i = CHUNKS_PER_TILE - 1
    pltpu.make_async_copy(g_sh.at[row_v.at[i]], rows0, sem0).wait()
    _scale(i, rows0)
    pltpu.async_copy(rows0, s_sh.at[col_v.at[i]], ssem0, add=True)
    pltpu.make_async_copy(rows1, s_sh.at[col_v.at[0]], ssem1).wait()
    pltpu.make_async_copy(rows0, s_sh.at[col_v.at[0]], ssem0).wait()

    plsc.subcore_barrier()
    pltpu.sync_copy(s_sh.at[pl.ds(s * NPT, NPT)],
                    sp_hbm.at[c, pl.ds(s * NPT, NPT)])


def _tc1_body(x_ref, w_ref, h_ref):
    h_ref[...] = jnp.dot(x_ref[...], w_ref[...],
                         preferred_element_type=jnp.float32)


def _tc1(x, W):
    return pl.pallas_call(
        _tc1_body,
        grid=(NP // BM,),
        in_specs=[
            pl.BlockSpec((BM, F), lambda i: (i, 0)),
            pl.BlockSpec((F, C), lambda i: (0, 0)),
        ],
        out_specs=pl.BlockSpec((BM, C), lambda i: (i, 0)),
        out_shape=jax.ShapeDtypeStruct((NP, C), jnp.float32),
    )(x, W)


def _tc2_body(sp_ref, h_ref, degp_ref, b_ref, out_ref):
    dinv = lax.rsqrt(1.0 + degp_ref[0] + degp_ref[1])
    t = sp_ref[0] + sp_ref[1] - h_ref[...] * dinv[:, None]
    z = t * dinv[:, None] + b_ref[...]
    out_ref[...] = jax.nn.sigmoid(z)


def _tc2(sp, h, degp, b):
    return pl.pallas_call(
        _tc2_body,
        grid=(NP // BM,),
        in_specs=[
            pl.BlockSpec((NC, BM, C), lambda i: (0, i, 0)),
            pl.BlockSpec((BM, C), lambda i: (i, 0)),
            pl.BlockSpec((NC, BM), lambda i: (0, i)),
            pl.BlockSpec((1, C), lambda i: (0, 0)),
        ],
        out_specs=pl.BlockSpec((BM, C), lambda i: (i, 0)),
        out_shape=jax.ShapeDtypeStruct((N, C), jnp.float32),
    )(sp, h, degp, b)


@jax.jit
def kernel(x, edge_index, edge_weight, W, b):
    row = edge_index[0].reshape(NW, CHUNKS_PER_TILE, K)
    col = edge_index[1].reshape(NW, CHUNKS_PER_TILE, K)
    ew = edge_weight.reshape(NW, CHUNKS_PER_TILE, K)

    degp = _deg_kernel(col, ew)
    h = _tc1(x, W)
    sp = _edge_kernel(row, col, ew, h, degp)
    out = _tc2(sp, h, degp, b.reshape(1, C))
    return out


# speedup vs baseline: 1.4076x; 1.1179x over previous
import functools

import jax
import jax.numpy as jnp
from jax import lax
from jax.experimental import pallas as pl
from jax.experimental.pallas import tpu as pltpu
from jax.experimental.pallas import tpu_sc as plsc

N = 10000
E = 320000
F = 128
C = 16

NC = 2
NS = 16
L = 16
NW = NC * NS

K = 80
CHUNKS_PER_TILE = E // (NW * K)
NP = 10240
NPT = NP // NS

BN = 512
GRID = -(-N // BN)
BM = 2048

_mesh = plsc.VectorSubcoreMesh(core_axis_name="c", subcore_axis_name="s")

_GDN = lax.GatherDimensionNumbers(
    offset_dims=(), collapsed_slice_dims=(0,), start_index_map=(0,))


def _take16(vec, k):
    idx = jnp.zeros((L, 1), jnp.int32) + k
    return lax.gather(vec, idx, dimension_numbers=_GDN,
                      slice_sizes=(1,),
                      mode=lax.GatherScatterMode.PROMISE_IN_BOUNDS)


@functools.partial(
    pl.kernel,
    out_type=jax.ShapeDtypeStruct((NC, NP), jnp.float32),
    mesh=_mesh,
    compiler_params=pltpu.CompilerParams(use_tc_tiling_on_sc=False),
    scratch_types=[
        pltpu.VMEM((CHUNKS_PER_TILE, K), jnp.int32),
        pltpu.VMEM((CHUNKS_PER_TILE, K), jnp.float32),
        pltpu.VMEM((NPT,), jnp.float32),
        pltpu.VMEM_SHARED((NP,), jnp.float32),
        pltpu.SemaphoreType.DMA,
    ],
)
def _deg_kernel(ei_hbm, ew_hbm, degp_hbm, col_v, ew_v, zbuf, deg_sh, dsem):
    c = lax.axis_index("c")
    s = lax.axis_index("s")
    wid = c * NS + s

    pltpu.sync_copy(ei_hbm.at[1, wid], col_v)
    pltpu.sync_copy(ew_hbm.at[wid], ew_v)

    def zero_body(i, _):
        zbuf[pl.ds(i * L, L)] = jnp.zeros((L,), jnp.float32)
        return 0
    lax.fori_loop(0, NPT // L, zero_body, 0)
    pltpu.sync_copy(zbuf, deg_sh.at[pl.ds(s * NPT, NPT)])
    plsc.subcore_barrier()

    def chunk_body(i, _):
        pltpu.async_copy(ew_v.at[i], deg_sh.at[col_v.at[i]], dsem, add=True)
        return 0
    lax.fori_loop(0, CHUNKS_PER_TILE, chunk_body, 0)

    def drain_body(i, _):
        pltpu.make_async_copy(ew_v.at[0], deg_sh.at[col_v.at[0]], dsem).wait()
        return 0
    lax.fori_loop(0, CHUNKS_PER_TILE, drain_body, 0)

    plsc.subcore_barrier()
    pltpu.sync_copy(deg_sh.at[pl.ds(s * NPT, NPT)],
                    degp_hbm.at[c, pl.ds(s * NPT, NPT)])


def _rsqrt16(d):
    i = lax.bitcast_convert_type(d, jnp.int32)
    i = 0x5F3759DF - lax.shift_right_logical(i, 1)
    y = lax.bitcast_convert_type(i, jnp.float32)
    for _ in range(3):
        y = y * (1.5 - 0.5 * d * y * y)
    return y


@functools.partial(
    pl.kernel,
    out_type=jax.ShapeDtypeStruct((NC, NP, C), jnp.float32),
    mesh=_mesh,
    compiler_params=pltpu.CompilerParams(use_tc_tiling_on_sc=False),
    scratch_types=[
        pltpu.VMEM((CHUNKS_PER_TILE, K), jnp.int32),
        pltpu.VMEM((CHUNKS_PER_TILE, K), jnp.int32),
        pltpu.VMEM((CHUNKS_PER_TILE, K), jnp.float32),
        pltpu.VMEM((NPT, C), jnp.float32),
        pltpu.VMEM((NPT,), jnp.float32),
        pltpu.VMEM((NPT,), jnp.float32),
        pltpu.VMEM((K, C), jnp.float32),
        pltpu.VMEM((K, C), jnp.float32),
        pltpu.VMEM_SHARED((NP, C), jnp.float32),
        pltpu.VMEM_SHARED((NP, C), jnp.float32),
        pltpu.SemaphoreType.DMA,
        pltpu.SemaphoreType.DMA,
        pltpu.SemaphoreType.DMA,
        pltpu.SemaphoreType.DMA,
    ],
)
def _edge_kernel(ei_hbm, ew_hbm, h_hbm, degp_hbm, sp_hbm,
                 row_v, col_v, ew_v, gbuf, d0_v, d1_v, rows0, rows1,
                 g_sh, s_sh, sem0, sem1, ssem0, ssem1):
    c = lax.axis_index("c")
    s = lax.axis_index("s")
    wid = c * NS + s

    pltpu.sync_copy(ei_hbm.at[0, wid], row_v)
    pltpu.sync_copy(ei_hbm.at[1, wid], col_v)
    pltpu.sync_copy(ew_hbm.at[wid], ew_v)

    pltpu.sync_copy(h_hbm.at[pl.ds(s * NPT, NPT)], gbuf)
    pltpu.sync_copy(degp_hbm.at[0, pl.ds(s * NPT, NPT)], d0_v)
    pltpu.sync_copy(degp_hbm.at[1, pl.ds(s * NPT, NPT)], d1_v)

    def g_body(v, _):
        d = 1.0 + d0_v[pl.ds(v * L, L)] + d1_v[pl.ds(v * L, L)]
        dv = _rsqrt16(d)
        for k2 in range(L):
            n = v * L + k2
            gbuf[n, :] = gbuf[n, :] * _take16(dv, k2)
        return 0
    lax.fori_loop(0, NPT // L, g_body, 0)

    pltpu.sync_copy(gbuf, g_sh.at[pl.ds(s * NPT, NPT)])
    pltpu.sync_copy(gbuf, s_sh.at[pl.ds(s * NPT, NPT)])
    plsc.subcore_barrier()

    rows = (rows0, rows1)
    sems = (sem0, sem1)
    ssems = (ssem0, ssem1)

    def _scale(i, buf):
        for j in range(K // L):
            ewv = ew_v[i, pl.ds(j * L, L)]
            for k2 in range(L):
                e = j * L + k2
                buf[e, :] = buf[e, :] * _take16(ewv, k2)

    pltpu.async_copy(g_sh.at[row_v.at[0]], rows0, sem0)
    pltpu.async_copy(g_sh.at[row_v.at[1]], rows1, sem1)
    pltpu.make_async_copy(g_sh.at[row_v.at[0]], rows0, sem0).wait()
    _scale(0, rows0)
    pltpu.async_copy(rows0, s_sh.at[col_v.at[0]], ssem0, add=True)

    def pair_body(i2, _):
        for b in (1, 0):
            i = i2 * 2 + (2 - b)
            ob = 1 - b
            pltpu.make_async_copy(rows[ob], s_sh.at[col_v.at[0]], ssems[ob]).wait()
            pltpu.async_copy(g_sh.at[row_v.at[i + 1]], rows[ob], sems[ob])
            pltpu.make_async_copy(g_sh.at[row_v.at[i]], rows[b], sems[b]).wait()
            _scale(i, rows[b])
            pltpu.async_copy(rows[b], s_sh.at[col_v.at[i]], ssems[b], add=True)
        return 0
    lax.fori_loop(0, (CHUNKS_PER_TILE - 3) // 2, pair_body, 0)

    i = CHUNKS_PER_TILE - 2
    pltpu.make_async_copy(rows0, s_sh.at[col_v.at[0]], ssem0).wait()
    pltpu.async_copy(g_sh.at[row_v.at[i + 1]], rows0, sem0)
    pltpu.make_async_copy(g_sh.at[row_v.at[i]], rows1, sem1).wait()
    _scale(i, rows1)
    pltpu.async_copy(rows1, s_sh.at[col_v.at[i]], ssem1, add=True)
    i = CHUNKS_PER_TILE - 1
    pltpu.make_async_copy(g_sh.at[row_v.at[i]], rows0, sem0).wait()
    _scale(i, rows0)
    pltpu.async_copy(rows0, s_sh.at[col_v.at[i]], ssem0, add=True)
    pltpu.make_async_copy(rows1, s_sh.at[col_v.at[0]], ssem1).wait()
    pltpu.make_async_copy(rows0, s_sh.at[col_v.at[0]], ssem0).wait()

    plsc.subcore_barrier()
    pltpu.sync_copy(s_sh.at[pl.ds(s * NPT, NPT)],
                    sp_hbm.at[c, pl.ds(s * NPT, NPT)])


def _tc1_body(x_ref, w_ref, h_ref):
    h_ref[...] = jnp.dot(x_ref[...], w_ref[...],
                         preferred_element_type=jnp.float32)


def _tc1(x, W):
    return pl.pallas_call(
        _tc1_body,
        grid=(NP // BM,),
        in_specs=[
            pl.BlockSpec((BM, F), lambda i: (i, 0)),
            pl.BlockSpec((F, C), lambda i: (0, 0)),
        ],
        out_specs=pl.BlockSpec((BM, C), lambda i: (i, 0)),
        out_shape=jax.ShapeDtypeStruct((NP, C), jnp.float32),
    )(x, W)


def _tc2_body(sp_ref, h_ref, degp_ref, b_ref, out_ref):
    dinv = lax.rsqrt(1.0 + degp_ref[0] + degp_ref[1])
    t = sp_ref[0] + sp_ref[1] - h_ref[...] * dinv[:, None]
    z = t * dinv[:, None] + b_ref[...]
    out_ref[...] = jax.nn.sigmoid(z)


def _tc2(sp, h, degp, b):
    return pl.pallas_call(
        _tc2_body,
        grid=(NP // BM,),
        in_specs=[
            pl.BlockSpec((NC, BM, C), lambda i: (0, i, 0)),
            pl.BlockSpec((BM, C), lambda i: (i, 0)),
            pl.BlockSpec((NC, BM), lambda i: (0, i)),
            pl.BlockSpec((1, C), lambda i: (0, 0)),
        ],
        out_specs=pl.BlockSpec((BM, C), lambda i: (i, 0)),
        out_shape=jax.ShapeDtypeStruct((N, C), jnp.float32),
    )(sp, h, degp, b)


@jax.jit
def kernel(x, edge_index, edge_weight, W, b):
    ei = edge_index.reshape(2, NW, CHUNKS_PER_TILE, K)
    ew = edge_weight.reshape(NW, CHUNKS_PER_TILE, K)

    degp = _deg_kernel(ei, ew)
    h = _tc1(x, W)
    sp = _edge_kernel(ei, ew, h, degp)
    out = _tc2(sp, h, degp, b.reshape(1, C))
    return out


# speedup vs baseline: 1.4273x; 1.0140x over previous
import functools

import jax
import jax.numpy as jnp
from jax import lax
from jax.experimental import pallas as pl
from jax.experimental.pallas import tpu as pltpu
from jax.experimental.pallas import tpu_sc as plsc

N = 10000
E = 320000
F = 128
C = 16

NC = 2
NS = 16
L = 16
NW = NC * NS

K = 80
CHUNKS_PER_TILE = E // (NW * K)
NP = 10240
NPT = NP // NS

BN = 512
GRID = -(-N // BN)
BM = 10240

_mesh = plsc.VectorSubcoreMesh(core_axis_name="c", subcore_axis_name="s")

_GDN = lax.GatherDimensionNumbers(
    offset_dims=(), collapsed_slice_dims=(0,), start_index_map=(0,))


def _take16(vec, k):
    idx = jnp.zeros((L, 1), jnp.int32) + k
    return lax.gather(vec, idx, dimension_numbers=_GDN,
                      slice_sizes=(1,),
                      mode=lax.GatherScatterMode.PROMISE_IN_BOUNDS)


@functools.partial(
    pl.kernel,
    out_type=jax.ShapeDtypeStruct((NC, NP), jnp.float32),
    mesh=_mesh,
    compiler_params=pltpu.CompilerParams(use_tc_tiling_on_sc=False),
    scratch_types=[
        pltpu.VMEM((CHUNKS_PER_TILE, K), jnp.int32),
        pltpu.VMEM((CHUNKS_PER_TILE, K), jnp.float32),
        pltpu.VMEM((NPT,), jnp.float32),
        pltpu.VMEM_SHARED((NP,), jnp.float32),
        pltpu.SemaphoreType.DMA,
    ],
)
def _deg_kernel(ei_hbm, ew_hbm, degp_hbm, col_v, ew_v, zbuf, deg_sh, dsem):
    c = lax.axis_index("c")
    s = lax.axis_index("s")
    wid = c * NS + s

    pltpu.sync_copy(ei_hbm.at[1, wid], col_v)
    pltpu.sync_copy(ew_hbm.at[wid], ew_v)

    def zero_body(i, _):
        zbuf[pl.ds(i * L, L)] = jnp.zeros((L,), jnp.float32)
        return 0
    lax.fori_loop(0, NPT // L, zero_body, 0)
    pltpu.sync_copy(zbuf, deg_sh.at[pl.ds(s * NPT, NPT)])
    plsc.subcore_barrier()

    def chunk_body(i, _):
        pltpu.async_copy(ew_v.at[i], deg_sh.at[col_v.at[i]], dsem, add=True)
        return 0
    lax.fori_loop(0, CHUNKS_PER_TILE, chunk_body, 0)

    def drain_body(i, _):
        pltpu.make_async_copy(ew_v.at[0], deg_sh.at[col_v.at[0]], dsem).wait()
        return 0
    lax.fori_loop(0, CHUNKS_PER_TILE, drain_body, 0)

    plsc.subcore_barrier()
    pltpu.sync_copy(deg_sh.at[pl.ds(s * NPT, NPT)],
                    degp_hbm.at[c, pl.ds(s * NPT, NPT)])


def _rsqrt16(d):
    i = lax.bitcast_convert_type(d, jnp.int32)
    i = 0x5F3759DF - lax.shift_right_logical(i, 1)
    y = lax.bitcast_convert_type(i, jnp.float32)
    for _ in range(3):
        y = y * (1.5 - 0.5 * d * y * y)
    return y


@functools.partial(
    pl.kernel,
    out_type=jax.ShapeDtypeStruct((NC, NP, C), jnp.float32),
    mesh=_mesh,
    compiler_params=pltpu.CompilerParams(use_tc_tiling_on_sc=False),
    scratch_types=[
        pltpu.VMEM((CHUNKS_PER_TILE, K), jnp.int32),
        pltpu.VMEM((CHUNKS_PER_TILE, K), jnp.int32),
        pltpu.VMEM((CHUNKS_PER_TILE, K), jnp.float32),
        pltpu.VMEM((NPT, C), jnp.float32),
        pltpu.VMEM((NPT,), jnp.float32),
        pltpu.VMEM((NPT,), jnp.float32),
        pltpu.VMEM((K, C), jnp.float32),
        pltpu.VMEM((K, C), jnp.float32),
        pltpu.VMEM_SHARED((NP, C), jnp.float32),
        pltpu.VMEM_SHARED((NP, C), jnp.float32),
        pltpu.SemaphoreType.DMA,
        pltpu.SemaphoreType.DMA,
        pltpu.SemaphoreType.DMA,
        pltpu.SemaphoreType.DMA,
    ],
)
def _edge_kernel(ei_hbm, ew_hbm, h_hbm, degp_hbm, sp_hbm,
                 row_v, col_v, ew_v, gbuf, d0_v, d1_v, rows0, rows1,
                 g_sh, s_sh, sem0, sem1, ssem0, ssem1):
    c = lax.axis_index("c")
    s = lax.axis_index("s")
    wid = c * NS + s

    pltpu.sync_copy(ei_hbm.at[0, wid], row_v)
    pltpu.sync_copy(ei_hbm.at[1, wid], col_v)
    pltpu.sync_copy(ew_hbm.at[wid], ew_v)

    pltpu.sync_copy(h_hbm.at[pl.ds(s * NPT, NPT)], gbuf)
    pltpu.sync_copy(degp_hbm.at[0, pl.ds(s * NPT, NPT)], d0_v)
    pltpu.sync_copy(degp_hbm.at[1, pl.ds(s * NPT, NPT)], d1_v)

    def g_body(v, _):
        d = 1.0 + d0_v[pl.ds(v * L, L)] + d1_v[pl.ds(v * L, L)]
        dv = _rsqrt16(d)
        for k2 in range(L):
            n = v * L + k2
            gbuf[n, :] = gbuf[n, :] * _take16(dv, k2)
        return 0
    lax.fori_loop(0, NPT // L, g_body, 0)

    pltpu.sync_copy(gbuf, g_sh.at[pl.ds(s * NPT, NPT)])
    pltpu.sync_copy(gbuf, s_sh.at[pl.ds(s * NPT, NPT)])
    plsc.subcore_barrier()

    rows = (rows0, rows1)
    sems = (sem0, sem1)
    ssems = (ssem0, ssem1)

    def _scale(i, buf):
        for j in range(K // L):
            ewv = ew_v[i, pl.ds(j * L, L)]
            for k2 in range(L):
                e = j * L + k2
                buf[e, :] = buf[e, :] * _take16(ewv, k2)

    pltpu.async_copy(g_sh.at[row_v.at[0]], rows0, sem0)
    pltpu.async_copy(g_sh.at[row_v.at[1]], rows1, sem1)
    pltpu.make_async_copy(g_sh.at[row_v.at[0]], rows0, sem0).wait()
    _scale(0, rows0)
    pltpu.async_copy(rows0, s_sh.at[col_v.at[0]], ssem0, add=True)

    def pair_body(i2, _):
        for b in (1, 0):
            i = i2 * 2 + (2 - b)
            ob = 1 - b
            pltpu.make_async_copy(rows[ob], s_sh.at[col_v.at[0]], ssems[ob]).wait()
            pltpu.async_copy(g_sh.at[row_v.at[i + 1]], rows[ob], sems[ob])
            pltpu.make_async_copy(g_sh.at[row_v.at[i]], rows[b], sems[b]).wait()
            _scale(i, rows[b])
            pltpu.async_copy(rows[b], s_sh.at[col_v.at[i]], ssems[b], add=True)
        return 0
    lax.fori_loop(0, (CHUNKS_PER_TILE - 3) // 2, pair_body, 0)

    i = CHUNKS_PER_TILE - 2
    pltpu.make_async_copy(rows0, s_sh.at[col_v.at[0]], ssem0).wait()
    pltpu.async_copy(g_sh.at[row_v.at[i + 1]], rows0, sem0)
    pltpu.make_async_copy(g_sh.at[row_v.at[i]], rows1, sem1).wait()
    _scale(i, rows1)
    pltpu.async_copy(rows1, s_sh.at[col_v.at[i]], ssem1, add=True)
    i = CHUNKS_PER_TILE - 1
    pltpu.make_async_copy(g_sh.at[row_v.at[i]], rows0, sem0).wait()
    _scale(i, rows0)
    pltpu.async_copy(rows0, s_sh.at[col_v.at[i]], ssem0, add=True)
    pltpu.make_async_copy(rows1, s_sh.at[col_v.at[0]], ssem1).wait()
    pltpu.make_async_copy(rows0, s_sh.at[col_v.at[0]], ssem0).wait()

    plsc.subcore_barrier()
    pltpu.sync_copy(s_sh.at[pl.ds(s * NPT, NPT)],
                    sp_hbm.at[c, pl.ds(s * NPT, NPT)])


def _tc1_body(x_ref, w_ref, h_ref):
    h_ref[...] = jnp.dot(x_ref[...], w_ref[...],
                         preferred_element_type=jnp.float32)


def _tc1(x, W):
    return pl.pallas_call(
        _tc1_body,
        grid=(NP // BM,),
        in_specs=[
            pl.BlockSpec((BM, F), lambda i: (i, 0)),
            pl.BlockSpec((F, C), lambda i: (0, 0)),
        ],
        out_specs=pl.BlockSpec((BM, C), lambda i: (i, 0)),
        out_shape=jax.ShapeDtypeStruct((NP, C), jnp.float32),
    )(x, W)


def _tc2_body(sp_ref, h_ref, degp_ref, b_ref, out_ref):
    dinv = lax.rsqrt(1.0 + degp_ref[0] + degp_ref[1])
    t = sp_ref[0] + sp_ref[1] - h_ref[...] * dinv[:, None]
    z = t * dinv[:, None] + b_ref[...]
    out_ref[...] = jax.nn.sigmoid(z)


def _tc2(sp, h, degp, b):
    return pl.pallas_call(
        _tc2_body,
        grid=(NP // BM,),
        in_specs=[
            pl.BlockSpec((NC, BM, C), lambda i: (0, i, 0)),
            pl.BlockSpec((BM, C), lambda i: (i, 0)),
            pl.BlockSpec((NC, BM), lambda i: (0, i)),
            pl.BlockSpec((1, C), lambda i: (0, 0)),
        ],
        out_specs=pl.BlockSpec((BM, C), lambda i: (i, 0)),
        out_shape=jax.ShapeDtypeStruct((N, C), jnp.float32),
    )(sp, h, degp, b)


@jax.jit
def kernel(x, edge_index, edge_weight, W, b):
    ei = edge_index.reshape(2, NW, CHUNKS_PER_TILE, K)
    ew = edge_weight.reshape(NW, CHUNKS_PER_TILE, K)

    degp = _deg_kernel(ei, ew)
    h = _tc1(x, W)
    sp = _edge_kernel(ei, ew, h, degp)
    out = _tc2(sp, h, degp, b.reshape(1, C))
    return out
